# baseline (speedup 1.0000x reference)
import dataclasses
import functools

import jax
import jax.numpy as jnp
from jax import lax
from jax.experimental import pallas as pl
from jax.experimental.pallas import tpu as pltpu
from jax.experimental.pallas import tpu_sc as plsc

_NC = 2
_NS = 16
_NW = _NC * _NS
_B = 128
_L = 16


def _compiler_params():
    cp = pltpu.CompilerParams()
    if "needs_layout_passes" in pltpu.CompilerParams.__dataclass_fields__:
        cp = dataclasses.replace(cp, needs_layout_passes=False)
    return cp


def _format_table(wT):
    emb, nv = wT.shape
    n_full = nv // _B
    mesh = plsc.VectorSubcoreMesh(core_axis_name="c", subcore_axis_name="s")

    @functools.partial(
        pl.kernel,
        mesh=mesh,
        compiler_params=_compiler_params(),
        out_type=jax.ShapeDtypeStruct((n_full * emb, 4 * emb), wT.dtype),
        scratch_types=[
            pltpu.VMEM((2, emb, _B + 8), jnp.float32),
            pltpu.VMEM((2, emb, _B), jnp.float32),
            pltpu.SemaphoreType.DMA,
            pltpu.SemaphoreType.DMA,
            pltpu.SemaphoreType.DMA,
            pltpu.SemaphoreType.DMA,
        ],
    )
    def fmt_kernel(wT_hbm, w4_hbm, in_v, out_v, rsem0, rsem1, wsem0, wsem1):
        wid = lax.axis_index("s") * _NC + lax.axis_index("c")
        rsem = (rsem0, rsem1)
        wsem = (wsem0, wsem1)
        fvec = (jax.lax.iota(jnp.int32, _L), jax.lax.iota(jnp.int32, _L) + _L)

        def blk_of(m):
            return wid + _NW * m

        def start_read(m, p):
            blk = blk_of(m)
            v0 = pl.multiple_of(blk * _B, _B)
            pltpu.make_async_copy(
                wT_hbm.at[pl.ds(0, emb), pl.ds(v0, _B)],
                in_v.at[p, pl.ds(0, emb), pl.ds(0, _B)], rsem[p]).start()

        def transpose_block(p, rows):
            for r in range(rows):
                for t in range(_B // _L):
                    col = jnp.full((_L,), 4 * r + (t >> 1), jnp.int32)
                    vals = plsc.load_gather(in_v.at[p], [fvec[t & 1], col])
                    out_v[p, r, pl.ds(t * _L, _L)] = vals

        def do_block(m, p):
            blk = blk_of(m)
            pltpu.make_async_copy(
                wT_hbm.at[pl.ds(0, emb), pl.ds(0, _B)],
                in_v.at[p, pl.ds(0, emb), pl.ds(0, _B)], rsem[p]).wait()

            @pl.when(m >= 2)
            def _():
                pltpu.make_async_copy(
                    out_v.at[p],
                    w4_hbm.at[pl.ds(0, emb), pl.ds(0, _B)],
                    wsem[p]).wait()

            transpose_block(p, emb)
            r0 = pl.multiple_of(blk * emb, 8)
            pltpu.make_async_copy(
                out_v.at[p],
                w4_hbm.at[pl.ds(r0, emb), pl.ds(0, _B)],
                wsem[p]).start()

        n_iter = n_full // _NW + 2
        start_read(0, 0)
        start_read(1, 1)

        @pl.loop(0, n_iter // 2)
        def _(g):
            for half in range(2):
                m = g * 2 + half
                p = half

                @pl.when(blk_of(m) < n_full)
                def _():
                    do_block(m, p)

                @pl.when(blk_of(m + 2) < n_full)
                def _():
                    start_read(m + 2, p)

        for p in range(2):
            pltpu.make_async_copy(
                out_v.at[p],
                w4_hbm.at[pl.ds(0, emb), pl.ds(0, _B)],
                wsem[p]).wait()

    return fmt_kernel(wT)


def _gather(xT, w4, w_tail, nj, nb, emb):
    chunks_per_j = nb // _B
    total_chunks = nj * chunks_per_j
    per_w = total_chunks // _NW
    n_lines = w4.shape[0]
    tail_lines = w_tail.shape[0]
    tail_base = n_lines * 4
    mesh = plsc.VectorSubcoreMesh(core_axis_name="c", subcore_axis_name="s")

    @functools.partial(
        pl.kernel,
        mesh=mesh,
        compiler_params=_compiler_params(),
        out_type=jax.ShapeDtypeStruct((nj, emb, nb), w4.dtype),
        scratch_types=[
            pltpu.VMEM((2, 8, _B), jnp.int32),
            pltpu.VMEM((2, _B), jnp.int32),
            pltpu.VMEM((2, _B), jnp.int32),
            pltpu.VMEM((2, _B, 4 * emb), jnp.float32),
            pltpu.VMEM((2, 1, emb, _B), jnp.float32),
            pltpu.VMEM((tail_lines, 4 * emb), jnp.float32),
            pltpu.SemaphoreType.DMA,
            pltpu.SemaphoreType.DMA,
            pltpu.SemaphoreType.DMA,
            pltpu.SemaphoreType.DMA,
            pltpu.SemaphoreType.DMA,
            pltpu.SemaphoreType.DMA,
        ],
    )
    def gather_kernel(xT_hbm, w4_hbm, wtail_hbm, out_hbm, idx_v, g_v, off_v,
                      gath_v, out_v, tail_v, isem0, isem1, gsem0, gsem1,
                      osem0, osem1):
        wid = lax.axis_index("s") * _NC + lax.axis_index("c")
        base = wid * per_w
        isem = (isem0, isem1)
        gsem = (gsem0, gsem1)
        osem = (osem0, osem1)

        def chunk_coords(i):
            cid = base + i
            j = cid >> 7
            b0 = pl.multiple_of((cid & (chunks_per_j - 1)) << 7, _B)
            j8 = pl.multiple_of((j >> 3) << 3, 8)
            return j, j8, b0

        def start_idx(i, p):
            _, j8, b0 = chunk_coords(i)
            pltpu.make_async_copy(
                xT_hbm.at[pl.ds(j8, 8), pl.ds(b0, _B)],
                idx_v.at[p], isem[p]).start()

        def compute_and_start_gather(i, p):
            j, _, _ = chunk_coords(i)
            jr = j & 7
            pltpu.make_async_copy(
                xT_hbm.at[pl.ds(0, 8), pl.ds(0, _B)],
                idx_v.at[p], isem[p]).wait()
            for s in range(_B // _L):
                raw = idx_v[p, jr, pl.ds(s * _L, _L)]
                g_v[p, pl.ds(s * _L, _L)] = jnp.minimum(raw >> 2, n_lines - 1)
                off_v[p, pl.ds(s * _L, _L)] = (raw & 3) << 5
            pltpu.make_async_copy(
                w4_hbm.at[g_v.at[p]], gath_v.at[p], gsem[p]).start()

        def extract_and_write(i, p):
            j, _, b0 = chunk_coords(i)
            jr = j & 7
            pltpu.make_async_copy(
                w4_hbm.at[g_v.at[p]], gath_v.at[p], gsem[p]).wait()
            for s in range(_B // _L):
                rows = jax.lax.iota(jnp.int32, _L) + s * _L
                cols0 = off_v[p, pl.ds(s * _L, _L)]
                for f in range(emb):
                    vals = plsc.load_gather(gath_v.at[p], [rows, cols0 + f])
                    out_v[p, 0, f, pl.ds(s * _L, _L)] = vals

            chunk_max = idx_v[p, jr, pl.ds(0, _L)]
            for s in range(1, _B // _L):
                chunk_max = jnp.maximum(chunk_max,
                                        idx_v[p, jr, pl.ds(s * _L, _L)])

            @pl.when(jnp.max(chunk_max) >= tail_base)
            def _():
                for s in range(_B // _L):
                    raw = idx_v[p, jr, pl.ds(s * _L, _L)]
                    m = raw >= tail_base
                    line = jnp.maximum((raw >> 2) - n_lines, 0)
                    cols0 = off_v[p, pl.ds(s * _L, _L)]
                    for f in range(emb):
                        tvals = plsc.load_gather(tail_v, [line, cols0 + f],
                                                 mask=m)
                        cur = out_v[p, 0, f, pl.ds(s * _L, _L)]
                        out_v[p, 0, f, pl.ds(s * _L, _L)] = jnp.where(
                            m, tvals, cur)

            @pl.when(i >= 2)
            def _():
                pltpu.make_async_copy(
                    out_v.at[p],
                    out_hbm.at[pl.ds(0, 1), pl.ds(0, emb), pl.ds(0, _B)],
                    osem[p]).wait()

            pltpu.make_async_copy(
                out_v.at[p],
                out_hbm.at[pl.ds(j, 1), pl.ds(0, emb), pl.ds(b0, _B)],
                osem[p]).start()

        pltpu.sync_copy(wtail_hbm, tail_v)
        start_idx(0, 0)
        start_idx(1, 1)
        compute_and_start_gather(0, 0)

        @pl.loop(0, per_w // 2)
        def _(g):
            for half in range(2):
                i = g * 2 + half
                p = half
                q = 1 - half

                @pl.when(i + 1 < per_w)
                def _():
                    compute_and_start_gather(i + 1, q)

                extract_and_write(i, p)

                @pl.when(i + 2 < per_w)
                def _():
                    start_idx(i + 2, p)

        for p in range(2):
            pltpu.make_async_copy(
                out_v.at[p],
                out_hbm.at[pl.ds(0, 1), pl.ds(0, emb), pl.ds(0, _B)],
                osem[p]).wait()

    return gather_kernel(xT, w4, w_tail)


def kernel(x, weight):
    nb, nj = x.shape
    nv, emb = weight.shape
    n_full = nv // _B
    w4 = _format_table(weight.T)
    w_tail = weight[n_full * _B:].reshape(-1, 4 * emb)
    outT = _gather(x.T, w4, w_tail, nj, nb, emb)
    return outT.transpose(2, 0, 1)


# speedup vs baseline: 1.1735x; 1.1735x over previous
import dataclasses
import functools

import jax
import jax.numpy as jnp
from jax import lax
from jax.experimental import pallas as pl
from jax.experimental.pallas import tpu as pltpu
from jax.experimental.pallas import tpu_sc as plsc

_NC = 2
_NS = 16
_NW = _NC * _NS
_B = 128
_L = 16


def _compiler_params():
    cp = pltpu.CompilerParams()
    if "needs_layout_passes" in pltpu.CompilerParams.__dataclass_fields__:
        cp = dataclasses.replace(cp, needs_layout_passes=False)
    return cp


def _format_table(wT):
    emb, nv = wT.shape
    n_full = nv // _B
    mesh = plsc.VectorSubcoreMesh(core_axis_name="c", subcore_axis_name="s")

    @functools.partial(
        pl.kernel,
        mesh=mesh,
        compiler_params=_compiler_params(),
        out_type=jax.ShapeDtypeStruct((n_full * emb, 4 * emb), wT.dtype),
        scratch_types=[
            pltpu.VMEM((2, emb, _B), jnp.float32),
            pltpu.VMEM((2, emb, _B), jnp.float32),
            pltpu.SemaphoreType.DMA,
            pltpu.SemaphoreType.DMA,
            pltpu.SemaphoreType.DMA,
            pltpu.SemaphoreType.DMA,
        ],
    )
    def fmt_kernel(wT_hbm, w4_hbm, in_v, out_v, rsem0, rsem1, wsem0, wsem1):
        wid = lax.axis_index("s") * _NC + lax.axis_index("c")
        rsem = (rsem0, rsem1)
        wsem = (wsem0, wsem1)
        fvec = (jax.lax.iota(jnp.int32, _L), jax.lax.iota(jnp.int32, _L) + _L)

        def blk_of(m):
            return wid + _NW * m

        def start_read(m, p):
            blk = blk_of(m)
            v0 = pl.multiple_of(blk * _B, _B)
            pltpu.make_async_copy(
                wT_hbm.at[pl.ds(0, emb), pl.ds(v0, _B)],
                in_v.at[p], rsem[p]).start()

        def transpose_block(p, rows):
            @pl.loop(0, rows)
            def _(r):
                for t in range(_B // _L):
                    col = jnp.full((_L,), 4 * r + (t >> 1), jnp.int32)
                    vals = plsc.load_gather(in_v.at[p], [fvec[t & 1], col])
                    out_v[p, r, pl.ds(t * _L, _L)] = vals

        def do_block(m, p):
            blk = blk_of(m)
            pltpu.make_async_copy(
                wT_hbm.at[pl.ds(0, emb), pl.ds(0, _B)],
                in_v.at[p], rsem[p]).wait()

            @pl.when(m >= 2)
            def _():
                pltpu.make_async_copy(
                    out_v.at[p],
                    w4_hbm.at[pl.ds(0, emb), pl.ds(0, _B)],
                    wsem[p]).wait()

            transpose_block(p, emb)
            r0 = pl.multiple_of(blk * emb, 8)
            pltpu.make_async_copy(
                out_v.at[p],
                w4_hbm.at[pl.ds(r0, emb), pl.ds(0, _B)],
                wsem[p]).start()

        n_iter = n_full // _NW + 2
        start_read(0, 0)
        start_read(1, 1)

        @pl.loop(0, n_iter // 2)
        def _(g):
            for half in range(2):
                m = g * 2 + half
                p = half

                @pl.when(blk_of(m) < n_full)
                def _():
                    do_block(m, p)

                @pl.when(blk_of(m + 2) < n_full)
                def _():
                    start_read(m + 2, p)

        for p in range(2):
            pltpu.make_async_copy(
                out_v.at[p],
                w4_hbm.at[pl.ds(0, emb), pl.ds(0, _B)],
                wsem[p]).wait()

    return fmt_kernel(wT)


def _gather(xT, w4, w_tail, nj, nb, emb):
    chunks_per_j = nb // _B
    total_chunks = nj * chunks_per_j
    per_w = total_chunks // _NW
    n_lines = w4.shape[0]
    tail_lines = w_tail.shape[0]
    tail_base = n_lines * 4
    mesh = plsc.VectorSubcoreMesh(core_axis_name="c", subcore_axis_name="s")

    @functools.partial(
        pl.kernel,
        mesh=mesh,
        compiler_params=_compiler_params(),
        out_type=jax.ShapeDtypeStruct((nj, emb, nb), w4.dtype),
        scratch_types=[
            pltpu.VMEM((2, 8, _B), jnp.int32),
            pltpu.VMEM((2, _B), jnp.int32),
            pltpu.VMEM((2, _B), jnp.int32),
            pltpu.VMEM((2, _B, 4 * emb), jnp.float32),
            pltpu.VMEM((2, 1, emb, _B), jnp.float32),
            pltpu.VMEM((tail_lines, 4 * emb), jnp.float32),
            pltpu.SemaphoreType.DMA,
            pltpu.SemaphoreType.DMA,
            pltpu.SemaphoreType.DMA,
            pltpu.SemaphoreType.DMA,
            pltpu.SemaphoreType.DMA,
            pltpu.SemaphoreType.DMA,
        ],
    )
    def gather_kernel(xT_hbm, w4_hbm, wtail_hbm, out_hbm, idx_v, g_v, off_v,
                      gath_v, out_v, tail_v, isem0, isem1, gsem0, gsem1,
                      osem0, osem1):
        wid = lax.axis_index("s") * _NC + lax.axis_index("c")
        base = wid * per_w
        isem = (isem0, isem1)
        gsem = (gsem0, gsem1)
        osem = (osem0, osem1)

        def chunk_coords(i):
            cid = base + i
            j = cid >> 7
            b0 = pl.multiple_of((cid & (chunks_per_j - 1)) << 7, _B)
            j8 = pl.multiple_of((j >> 3) << 3, 8)
            return j, j8, b0

        def start_idx(i, p):
            _, j8, b0 = chunk_coords(i)
            pltpu.make_async_copy(
                xT_hbm.at[pl.ds(j8, 8), pl.ds(b0, _B)],
                idx_v.at[p], isem[p]).start()

        def compute_and_start_gather(i, p):
            j, _, _ = chunk_coords(i)
            jr = j & 7
            pltpu.make_async_copy(
                xT_hbm.at[pl.ds(0, 8), pl.ds(0, _B)],
                idx_v.at[p], isem[p]).wait()
            for s in range(_B // _L):
                raw = idx_v[p, jr, pl.ds(s * _L, _L)]
                g_v[p, pl.ds(s * _L, _L)] = jnp.minimum(raw >> 2, n_lines - 1)
                off_v[p, pl.ds(s * _L, _L)] = (raw & 3) << 5
            pltpu.make_async_copy(
                w4_hbm.at[g_v.at[p]], gath_v.at[p], gsem[p]).start()

        def extract_and_write(i, p):
            j, _, b0 = chunk_coords(i)
            jr = j & 7
            pltpu.make_async_copy(
                w4_hbm.at[g_v.at[p]], gath_v.at[p], gsem[p]).wait()
            @pl.loop(0, _B // _L)
            def _(s):
                rows = jax.lax.iota(jnp.int32, _L) + s * _L
                cols0 = off_v[p, pl.ds(s * _L, _L)]
                for f in range(emb):
                    vals = plsc.load_gather(gath_v.at[p], [rows, cols0 + f])
                    out_v[p, 0, f, pl.ds(s * _L, _L)] = vals

            chunk_max = idx_v[p, jr, pl.ds(0, _L)]
            for s in range(1, _B // _L):
                chunk_max = jnp.maximum(chunk_max,
                                        idx_v[p, jr, pl.ds(s * _L, _L)])

            @pl.when(jnp.max(chunk_max) >= tail_base)
            def _():
                @pl.loop(0, _B // _L)
                def _(s):
                    raw = idx_v[p, jr, pl.ds(s * _L, _L)]
                    m = raw >= tail_base
                    line = jnp.maximum((raw >> 2) - n_lines, 0)
                    cols0 = off_v[p, pl.ds(s * _L, _L)]
                    for f in range(emb):
                        tvals = plsc.load_gather(tail_v, [line, cols0 + f],
                                                 mask=m)
                        cur = out_v[p, 0, f, pl.ds(s * _L, _L)]
                        out_v[p, 0, f, pl.ds(s * _L, _L)] = jnp.where(
                            m, tvals, cur)

            @pl.when(i >= 2)
            def _():
                pltpu.make_async_copy(
                    out_v.at[p],
                    out_hbm.at[pl.ds(0, 1), pl.ds(0, emb), pl.ds(0, _B)],
                    osem[p]).wait()

            pltpu.make_async_copy(
                out_v.at[p],
                out_hbm.at[pl.ds(j, 1), pl.ds(0, emb), pl.ds(b0, _B)],
                osem[p]).start()

        pltpu.sync_copy(wtail_hbm, tail_v)
        start_idx(0, 0)
        start_idx(1, 1)
        compute_and_start_gather(0, 0)

        @pl.loop(0, per_w // 2)
        def _(g):
            for half in range(2):
                i = g * 2 + half
                p = half
                q = 1 - half

                @pl.when(i + 1 < per_w)
                def _():
                    compute_and_start_gather(i + 1, q)

                extract_and_write(i, p)

                @pl.when(i + 2 < per_w)
                def _():
                    start_idx(i + 2, p)

        for p in range(2):
            pltpu.make_async_copy(
                out_v.at[p],
                out_hbm.at[pl.ds(0, 1), pl.ds(0, emb), pl.ds(0, _B)],
                osem[p]).wait()

    return gather_kernel(xT, w4, w_tail)


def kernel(x, weight):
    nb, nj = x.shape
    nv, emb = weight.shape
    n_full = nv // _B
    w4 = _format_table(weight.T)
    w_tail = weight[n_full * _B:].reshape(-1, 4 * emb)
    outT = _gather(x.T, w4, w_tail, nj, nb, emb)
    return outT.transpose(2, 0, 1)


# speedup vs baseline: 1.4404x; 1.2274x over previous
import dataclasses
import functools

import jax
import jax.numpy as jnp
from jax import lax
from jax.experimental import pallas as pl
from jax.experimental.pallas import tpu as pltpu
from jax.experimental.pallas import tpu_sc as plsc

_NC = 2
_NS = 16
_NW = _NC * _NS
_B = 128
_L = 16


def _compiler_params():
    cp = pltpu.CompilerParams()
    if "needs_layout_passes" in pltpu.CompilerParams.__dataclass_fields__:
        cp = dataclasses.replace(cp, needs_layout_passes=False)
    return cp


def _gather(xT, w4, nj, nb, emb):
    chunks_per_j = nb // _B
    total_chunks = nj * chunks_per_j
    per_w = total_chunks // _NW
    mesh = plsc.VectorSubcoreMesh(core_axis_name="c", subcore_axis_name="s")

    @functools.partial(
        pl.kernel,
        mesh=mesh,
        compiler_params=_compiler_params(),
        out_type=jax.ShapeDtypeStruct((nj, emb, nb), w4.dtype),
        scratch_types=[
            pltpu.VMEM((2, 8, _B), jnp.int32),
            pltpu.VMEM((2, _B), jnp.int32),
            pltpu.VMEM((2, _B), jnp.int32),
            pltpu.VMEM((2, _B, 4 * emb), jnp.float32),
            pltpu.VMEM((2, 1, emb, _B), jnp.float32),
            pltpu.SemaphoreType.DMA,
            pltpu.SemaphoreType.DMA,
            pltpu.SemaphoreType.DMA,
            pltpu.SemaphoreType.DMA,
            pltpu.SemaphoreType.DMA,
            pltpu.SemaphoreType.DMA,
        ],
    )
    def gather_kernel(xT_hbm, w4_hbm, out_hbm, idx_v, g_v, off_v, gath_v,
                      out_v, isem0, isem1, gsem0, gsem1, osem0, osem1):
        wid = lax.axis_index("s") * _NC + lax.axis_index("c")
        base = wid * per_w
        isem = (isem0, isem1)
        gsem = (gsem0, gsem1)
        osem = (osem0, osem1)

        def chunk_coords(i):
            cid = base + i
            j = cid >> 7
            b0 = pl.multiple_of((cid & (chunks_per_j - 1)) << 7, _B)
            j8 = pl.multiple_of((j >> 3) << 3, 8)
            return j, j8, b0

        def start_idx(i, p):
            _, j8, b0 = chunk_coords(i)
            pltpu.make_async_copy(
                xT_hbm.at[pl.ds(j8, 8), pl.ds(b0, _B)],
                idx_v.at[p], isem[p]).start()

        def compute_and_start_gather(i, p):
            j, _, _ = chunk_coords(i)
            jr = j & 7
            pltpu.make_async_copy(
                xT_hbm.at[pl.ds(0, 8), pl.ds(0, _B)],
                idx_v.at[p], isem[p]).wait()
            for s in range(_B // _L):
                raw = idx_v[p, jr, pl.ds(s * _L, _L)]
                g_v[p, pl.ds(s * _L, _L)] = raw >> 2
                off_v[p, pl.ds(s * _L, _L)] = (raw & 3) << 5
            pltpu.make_async_copy(
                w4_hbm.at[g_v.at[p]], gath_v.at[p], gsem[p]).start()

        def extract_and_write(i, p):
            j, _, b0 = chunk_coords(i)
            pltpu.make_async_copy(
                w4_hbm.at[g_v.at[p]], gath_v.at[p], gsem[p]).wait()

            @pl.loop(0, _B // _L)
            def _(s):
                rows = jax.lax.iota(jnp.int32, _L) + s * _L
                cols0 = off_v[p, pl.ds(s * _L, _L)]
                for f in range(emb):
                    vals = plsc.load_gather(gath_v.at[p], [rows, cols0 + f])
                    out_v[p, 0, f, pl.ds(s * _L, _L)] = vals

            @pl.when(i >= 2)
            def _():
                pltpu.make_async_copy(
                    out_v.at[p],
                    out_hbm.at[pl.ds(0, 1), pl.ds(0, emb), pl.ds(0, _B)],
                    osem[p]).wait()

            pltpu.make_async_copy(
                out_v.at[p],
                out_hbm.at[pl.ds(j, 1), pl.ds(0, emb), pl.ds(b0, _B)],
                osem[p]).start()

        start_idx(0, 0)
        start_idx(1, 1)
        compute_and_start_gather(0, 0)

        @pl.loop(0, per_w // 2)
        def _(g):
            for half in range(2):
                i = g * 2 + half
                p = half
                q = 1 - half

                @pl.when(i + 1 < per_w)
                def _():
                    compute_and_start_gather(i + 1, q)

                extract_and_write(i, p)

                @pl.when(i + 2 < per_w)
                def _():
                    start_idx(i + 2, p)

        for p in range(2):
            pltpu.make_async_copy(
                out_v.at[p],
                out_hbm.at[pl.ds(0, 1), pl.ds(0, emb), pl.ds(0, _B)],
                osem[p]).wait()

    return gather_kernel(xT, w4)


def kernel(x, weight):
    nb, nj = x.shape
    nv, emb = weight.shape
    w4 = weight.reshape(nv // 4, 4 * emb)
    outT = _gather(x.T, w4, nj, nb, emb)
    return outT.transpose(2, 0, 1)


# speedup vs baseline: 1.7997x; 1.2495x over previous
import dataclasses
import functools

import jax
import jax.numpy as jnp
from jax import lax
from jax.experimental import pallas as pl
from jax.experimental.pallas import tpu as pltpu
from jax.experimental.pallas import tpu_sc as plsc

_NC = 2
_NS = 16
_NW = _NC * _NS
_B = 128
_L = 16


def _compiler_params():
    cp = pltpu.CompilerParams()
    if "needs_layout_passes" in pltpu.CompilerParams.__dataclass_fields__:
        cp = dataclasses.replace(cp, needs_layout_passes=False)
    return cp


def _gather(xT, w4, nj, nb, emb):
    chunks_per_j = nb // _B
    total_chunks = nj * chunks_per_j
    per_w = total_chunks // _NW
    mesh = plsc.VectorSubcoreMesh(core_axis_name="c", subcore_axis_name="s")

    @functools.partial(
        pl.kernel,
        mesh=mesh,
        compiler_params=_compiler_params(),
        out_type=jax.ShapeDtypeStruct((nj, emb, nb), w4.dtype),
        scratch_types=[
            pltpu.VMEM((2, 8, _B), jnp.int32),
            pltpu.VMEM((2, _B), jnp.int32),
            pltpu.VMEM((2, _B), jnp.int32),
            pltpu.VMEM((2, _B, 4 * emb), jnp.float32),
            pltpu.VMEM((2, 1, emb, _B), jnp.float32),
            pltpu.SemaphoreType.DMA,
            pltpu.SemaphoreType.DMA,
            pltpu.SemaphoreType.DMA,
            pltpu.SemaphoreType.DMA,
            pltpu.SemaphoreType.DMA,
            pltpu.SemaphoreType.DMA,
        ],
    )
    def gather_kernel(xT_hbm, w4_hbm, out_hbm, idx_v, g_v, off_v, gath_v,
                      out_v, isem0, isem1, gsem0, gsem1, osem0, osem1):
        wid = lax.axis_index("s") * _NC + lax.axis_index("c")
        base = wid * per_w
        isem = (isem0, isem1)
        gsem = (gsem0, gsem1)
        osem = (osem0, osem1)

        def chunk_coords(i):
            cid = base + i
            j = cid >> 7
            b0 = pl.multiple_of((cid & (chunks_per_j - 1)) << 7, _B)
            j8 = pl.multiple_of((j >> 3) << 3, 8)
            return j, j8, b0

        def start_idx(i, p):
            _, j8, b0 = chunk_coords(i)
            pltpu.make_async_copy(
                xT_hbm.at[pl.ds(j8, 8), pl.ds(b0, _B)],
                idx_v.at[p], isem[p]).start()

        def compute_and_start_gather(i, p):
            j, _, _ = chunk_coords(i)
            jr = j & 7
            pltpu.make_async_copy(
                xT_hbm.at[pl.ds(0, 8), pl.ds(0, _B)],
                idx_v.at[p], isem[p]).wait()
            for s in range(_B // _L):
                raw = idx_v[p, jr, pl.ds(s * _L, _L)]
                g_v[p, pl.ds(s * _L, _L)] = raw >> 2
                off_v[p, pl.ds(s * _L, _L)] = (raw & 3) << 5
            pltpu.make_async_copy(
                w4_hbm.at[g_v.at[p]], gath_v.at[p], gsem[p]).start()

        def extract_and_write(i, p):
            j, _, b0 = chunk_coords(i)
            pltpu.make_async_copy(
                w4_hbm.at[g_v.at[p]], gath_v.at[p], gsem[p]).wait()

            @plsc.parallel_loop(0, _B // _L, unroll=2)
            def _(s):
                rows = jax.lax.iota(jnp.int32, _L) + s * _L
                cols0 = off_v[p, pl.ds(s * _L, _L)]
                for f in range(emb):
                    vals = plsc.load_gather(gath_v.at[p], [rows, cols0 + f])
                    out_v[p, 0, f, pl.ds(s * _L, _L)] = vals

            @pl.when(i >= 2)
            def _():
                pltpu.make_async_copy(
                    out_v.at[p],
                    out_hbm.at[pl.ds(0, 1), pl.ds(0, emb), pl.ds(0, _B)],
                    osem[p]).wait()

            pltpu.make_async_copy(
                out_v.at[p],
                out_hbm.at[pl.ds(j, 1), pl.ds(0, emb), pl.ds(b0, _B)],
                osem[p]).start()

        start_idx(0, 0)
        start_idx(1, 1)
        compute_and_start_gather(0, 0)

        @pl.loop(0, per_w // 2)
        def _(g):
            for half in range(2):
                i = g * 2 + half
                p = half
                q = 1 - half

                @pl.when(i + 1 < per_w)
                def _():
                    compute_and_start_gather(i + 1, q)

                extract_and_write(i, p)

                @pl.when(i + 2 < per_w)
                def _():
                    start_idx(i + 2, p)

        for p in range(2):
            pltpu.make_async_copy(
                out_v.at[p],
                out_hbm.at[pl.ds(0, 1), pl.ds(0, emb), pl.ds(0, _B)],
                osem[p]).wait()

    return gather_kernel(xT, w4)


def kernel(x, weight):
    nb, nj = x.shape
    nv, emb = weight.shape
    w4 = weight.reshape(nv // 4, 4 * emb)
    outT = _gather(x.T, w4, nj, nb, emb)
    return outT.transpose(2, 0, 1)


# speedup vs baseline: 1.9288x; 1.0717x over previous
import dataclasses
import functools

import jax
import jax.numpy as jnp
from jax import lax
from jax.experimental import pallas as pl
from jax.experimental.pallas import tpu as pltpu
from jax.experimental.pallas import tpu_sc as plsc

_NC = 2
_NS = 16
_NW = _NC * _NS
_B = 128
_L = 16


def _compiler_params():
    cp = pltpu.CompilerParams()
    if "needs_layout_passes" in pltpu.CompilerParams.__dataclass_fields__:
        cp = dataclasses.replace(cp, needs_layout_passes=False)
    return cp


def _format_table(wT):
    emb, nv = wT.shape
    n_full = nv // _B
    mesh = plsc.VectorSubcoreMesh(core_axis_name="c", subcore_axis_name="s")

    @functools.partial(
        pl.kernel,
        mesh=mesh,
        compiler_params=_compiler_params(),
        out_type=jax.ShapeDtypeStruct((n_full * emb, 4 * emb), wT.dtype),
        scratch_types=[
            pltpu.VMEM((2, emb, _B), jnp.float32),
            pltpu.VMEM((2, emb, _B), jnp.float32),
            pltpu.SemaphoreType.DMA,
            pltpu.SemaphoreType.DMA,
            pltpu.SemaphoreType.DMA,
            pltpu.SemaphoreType.DMA,
        ],
    )
    def fmt_kernel(wT_hbm, w4_hbm, in_v, out_v, rsem0, rsem1, wsem0, wsem1):
        wid = lax.axis_index("s") * _NC + lax.axis_index("c")
        rsem = (rsem0, rsem1)
        wsem = (wsem0, wsem1)
        fvec = (jax.lax.iota(jnp.int32, _L), jax.lax.iota(jnp.int32, _L) + _L)

        def blk_of(m):
            return wid + _NW * m

        def start_read(m, p):
            blk = blk_of(m)
            v0 = pl.multiple_of(blk * _B, _B)
            pltpu.make_async_copy(
                wT_hbm.at[pl.ds(0, emb), pl.ds(v0, _B)],
                in_v.at[p], rsem[p]).start()

        def do_block(m, p):
            blk = blk_of(m)
            pltpu.make_async_copy(
                wT_hbm.at[pl.ds(0, emb), pl.ds(0, _B)],
                in_v.at[p], rsem[p]).wait()

            @pl.when(m >= 2)
            def _():
                pltpu.make_async_copy(
                    out_v.at[p],
                    w4_hbm.at[pl.ds(0, emb), pl.ds(0, _B)],
                    wsem[p]).wait()

            @plsc.parallel_loop(0, emb, unroll=2)
            def _(r):
                for t in range(_B // _L):
                    col = jnp.full((_L,), 4 * r + (t >> 1), jnp.int32)
                    vals = plsc.load_gather(in_v.at[p], [fvec[t & 1], col])
                    out_v[p, r, pl.ds(t * _L, _L)] = vals

            r0 = pl.multiple_of(blk * emb, 8)
            pltpu.make_async_copy(
                out_v.at[p],
                w4_hbm.at[pl.ds(r0, emb), pl.ds(0, _B)],
                wsem[p]).start()

        n_iter = n_full // _NW + 2
        start_read(0, 0)
        start_read(1, 1)

        @pl.loop(0, n_iter // 2)
        def _(g):
            for half in range(2):
                m = g * 2 + half
                p = half

                @pl.when(blk_of(m) < n_full)
                def _():
                    do_block(m, p)

                @pl.when(blk_of(m + 2) < n_full)
                def _():
                    start_read(m + 2, p)

        for p in range(2):
            pltpu.make_async_copy(
                out_v.at[p],
                w4_hbm.at[pl.ds(0, emb), pl.ds(0, _B)],
                wsem[p]).wait()

    return fmt_kernel(wT)


def _gather(xT, w4, w_tail, nj, nb, emb):
    chunks_per_j = nb // _B
    total_chunks = nj * chunks_per_j
    per_w = total_chunks // _NW
    n_lines = w4.shape[0]
    tail_lines = w_tail.shape[0]
    tail_base = n_lines * 4
    mesh = plsc.VectorSubcoreMesh(core_axis_name="c", subcore_axis_name="s")

    @functools.partial(
        pl.kernel,
        mesh=mesh,
        compiler_params=_compiler_params(),
        out_type=jax.ShapeDtypeStruct((nj, emb, nb), w4.dtype),
        scratch_types=[
            pltpu.VMEM((2, 8, _B), jnp.int32),
            pltpu.VMEM((2, _B), jnp.int32),
            pltpu.VMEM((2, _B), jnp.int32),
            pltpu.VMEM((2, _B, 4 * emb), jnp.float32),
            pltpu.VMEM((2, 1, emb, _B), jnp.float32),
            pltpu.VMEM((16, 4 * emb), jnp.float32),
            pltpu.SemaphoreType.DMA,
            pltpu.SemaphoreType.DMA,
            pltpu.SemaphoreType.DMA,
            pltpu.SemaphoreType.DMA,
            pltpu.SemaphoreType.DMA,
            pltpu.SemaphoreType.DMA,
        ],
    )
    def gather_kernel(xT_hbm, w4_hbm, wtail_hbm, out_hbm, idx_v, g_v, off_v,
                      gath_v, out_v, tail_v, isem0, isem1, gsem0, gsem1,
                      osem0, osem1):
        wid = lax.axis_index("s") * _NC + lax.axis_index("c")
        base = wid * per_w
        isem = (isem0, isem1)
        gsem = (gsem0, gsem1)
        osem = (osem0, osem1)

        def chunk_coords(i):
            cid = base + i
            j = cid >> 7
            b0 = pl.multiple_of((cid & (chunks_per_j - 1)) << 7, _B)
            j8 = pl.multiple_of((j >> 3) << 3, 8)
            return j, j8, b0

        def start_idx(i, p):
            _, j8, b0 = chunk_coords(i)
            pltpu.make_async_copy(
                xT_hbm.at[pl.ds(j8, 8), pl.ds(b0, _B)],
                idx_v.at[p], isem[p]).start()

        def compute_and_start_gather(i, p):
            j, _, _ = chunk_coords(i)
            jr = j & 7
            pltpu.make_async_copy(
                xT_hbm.at[pl.ds(0, 8), pl.ds(0, _B)],
                idx_v.at[p], isem[p]).wait()
            for s in range(_B // _L):
                raw = idx_v[p, jr, pl.ds(s * _L, _L)]
                g_v[p, pl.ds(s * _L, _L)] = jnp.minimum(raw >> 2, n_lines - 1)
                off_v[p, pl.ds(s * _L, _L)] = (raw & 3) << 5
            pltpu.make_async_copy(
                w4_hbm.at[g_v.at[p]], gath_v.at[p], gsem[p]).start()

        def extract_and_write(i, p):
            j, _, b0 = chunk_coords(i)
            jr = j & 7
            pltpu.make_async_copy(
                w4_hbm.at[g_v.at[p]], gath_v.at[p], gsem[p]).wait()

            @plsc.parallel_loop(0, _B // _L, unroll=2)
            def _(s):
                rows = jax.lax.iota(jnp.int32, _L) + s * _L
                cols0 = off_v[p, pl.ds(s * _L, _L)]
                for f in range(emb):
                    vals = plsc.load_gather(gath_v.at[p], [rows, cols0 + f])
                    out_v[p, 0, f, pl.ds(s * _L, _L)] = vals

            chunk_max = idx_v[p, jr, pl.ds(0, _L)]
            for s in range(1, _B // _L):
                chunk_max = jnp.maximum(chunk_max,
                                        idx_v[p, jr, pl.ds(s * _L, _L)])

            @pl.when(jnp.max(chunk_max) >= tail_base)
            def _():
                @pl.loop(0, _B // _L)
                def _(s):
                    raw = idx_v[p, jr, pl.ds(s * _L, _L)]
                    m = raw >= tail_base
                    line = jnp.maximum((raw >> 2) - n_lines, 0)
                    cols0 = off_v[p, pl.ds(s * _L, _L)]
                    for f in range(emb):
                        tvals = plsc.load_gather(tail_v, [line, cols0 + f],
                                                 mask=m)
                        cur = out_v[p, 0, f, pl.ds(s * _L, _L)]
                        out_v[p, 0, f, pl.ds(s * _L, _L)] = jnp.where(
                            m, tvals, cur)

            @pl.when(i >= 2)
            def _():
                pltpu.make_async_copy(
                    out_v.at[p],
                    out_hbm.at[pl.ds(0, 1), pl.ds(0, emb), pl.ds(0, _B)],
                    osem[p]).wait()

            pltpu.make_async_copy(
                out_v.at[p],
                out_hbm.at[pl.ds(j, 1), pl.ds(0, emb), pl.ds(b0, _B)],
                osem[p]).start()

        pltpu.sync_copy(wtail_hbm, tail_v)
        start_idx(0, 0)
        start_idx(1, 1)
        compute_and_start_gather(0, 0)

        @pl.loop(0, per_w // 2)
        def _(g):
            for half in range(2):
                i = g * 2 + half
                p = half
                q = 1 - half

                @pl.when(i + 1 < per_w)
                def _():
                    compute_and_start_gather(i + 1, q)

                extract_and_write(i, p)

                @pl.when(i + 2 < per_w)
                def _():
                    start_idx(i + 2, p)

        for p in range(2):
            pltpu.make_async_copy(
                out_v.at[p],
                out_hbm.at[pl.ds(0, 1), pl.ds(0, emb), pl.ds(0, _B)],
                osem[p]).wait()

    return gather_kernel(xT, w4, w_tail)


def kernel(x, weight):
    nb, nj = x.shape
    nv, emb = weight.shape
    n_full = nv // _B
    w4 = _format_table(weight.T)
    w_tail = weight[n_full * _B:].reshape(-1, 4 * emb)
    outT = _gather(x.T, w4, w_tail, nj, nb, emb)
    return outT.transpose(2, 0, 1)
